# baseline (speedup 1.0000x reference)
import functools

import jax
import jax.numpy as jnp
from jax import lax
from jax.experimental import pallas as pl
from jax.experimental.pallas import tpu as pltpu
from jax.experimental.pallas import tpu_sc as plsc

B = 2
N = 4096
CH = 256
HEADS = 8
NR = 2
PATCH = 256
DH = 96
PADW = 128
EXTW = 128
BS = B * HEADS
TBL = BS * N

SC_CORES = 2
SC_SUBCORES = 16
NW = SC_CORES * SC_SUBCORES
CHUNK = 128
ROWS_PER_W = TBL // NW
NCHUNKS = ROWS_PER_W // CHUNK

BN = 512
NB = N // BN
ATT_B = 2
ATT_ROWS = ATT_B * PATCH


@functools.cache
def _sc_mesh():
    return plsc.VectorSubcoreMesh(
        core_axis_name="c", subcore_axis_name="s",
        num_cores=SC_CORES, num_subcores=SC_SUBCORES)


def _qkv_body(x_ref, wq_ref, wk_ref, wv_ref, qk_ref, v_ref):
    x = x_ref[0]
    xb = x.astype(jnp.bfloat16)
    q = jnp.dot(xb, wq_ref[0].astype(jnp.bfloat16),
                preferred_element_type=jnp.float32)
    k = jnp.dot(xb, wk_ref[0].astype(jnp.bfloat16),
                preferred_element_type=jnp.float32)
    qu = lax.bitcast_convert_type(q.astype(jnp.bfloat16), jnp.uint16)
    ku = lax.bitcast_convert_type(k.astype(jnp.bfloat16), jnp.uint16)
    word = qu.astype(jnp.uint32) | (ku.astype(jnp.uint32) << 16)
    qk_ref[0] = lax.bitcast_convert_type(word, jnp.int32)
    v_ref[0] = jnp.dot(x, wv_ref[0], preferred_element_type=jnp.float32)


def _qkv_tables(inp, Wq3, Wk3, Wv3):
    grid = (B, NB, HEADS)
    wspec = pl.BlockSpec((1, CH, PADW), lambda b, nb, h: (h, 0, 0))
    return pl.pallas_call(
        _qkv_body,
        grid=grid,
        in_specs=[
            pl.BlockSpec((1, BN, CH), lambda b, nb, h: (b, nb, 0)),
            wspec, wspec, wspec,
        ],
        out_specs=[
            pl.BlockSpec((1, BN, PADW), lambda b, nb, h: (b * HEADS + h, nb, 0)),
            pl.BlockSpec((1, BN, PADW), lambda b, nb, h: (b * HEADS + h, nb, 0)),
        ],
        out_shape=[
            jax.ShapeDtypeStruct((BS, N, PADW), jnp.int32),
            jax.ShapeDtypeStruct((BS, N, PADW), jnp.float32),
        ],
    )(inp, Wq3, Wk3, Wv3)


@functools.cache
def _sc_gather_kernel():
    @functools.partial(
        pl.kernel,
        out_type=[
            jax.ShapeDtypeStruct((TBL, PADW), jnp.int32),
            jax.ShapeDtypeStruct((TBL, PADW), jnp.float32),
        ],
        mesh=_sc_mesh(),
        scratch_types=[
            pltpu.VMEM((CHUNK,), jnp.int32),
            pltpu.VMEM((CHUNK, PADW), jnp.int32),
            pltpu.VMEM((CHUNK, PADW), jnp.float32),
            pltpu.SemaphoreType.DMA,
        ],
        cost_estimate=pl.CostEstimate(
            flops=0, transcendentals=0,
            bytes_accessed=2 * TBL * PADW * 8 + TBL * 4),
    )
    def body(idx_hbm, qkt_hbm, vt_hbm, sqk_hbm, sv_hbm,
             idx_v, bqk, bv, sem):
        wid = lax.axis_index("s") * SC_CORES + lax.axis_index("c")

        def step(i, carry):
            base = wid * ROWS_PER_W + i * CHUNK
            pltpu.sync_copy(idx_hbm.at[pl.ds(base, CHUNK)], idx_v)
            cqk = pltpu.async_copy(qkt_hbm.at[idx_v], bqk, sem)
            cv = pltpu.async_copy(vt_hbm.at[idx_v], bv, sem)
            cqk.wait()
            cv.wait()
            pltpu.sync_copy(bqk, sqk_hbm.at[pl.ds(base, CHUNK)])
            pltpu.sync_copy(bv, sv_hbm.at[pl.ds(base, CHUNK)])
            return carry

        lax.fori_loop(0, NCHUNKS, step, 0)

    return body


def _attn_body(qk_ref, v_ref, o_ref):
    lane = lax.broadcasted_iota(jnp.int32, (PATCH, EXTW), 1)
    for t in range(ATT_B):
        sl = pl.ds(t * PATCH, PATCH)
        w = lax.bitcast_convert_type(qk_ref[sl, :], jnp.uint32)
        q = lax.bitcast_convert_type((w & 0xFFFF).astype(jnp.uint16),
                                     jnp.bfloat16)
        k = lax.bitcast_convert_type((w >> 16).astype(jnp.uint16),
                                     jnp.bfloat16)
        v = v_ref[sl, :]
        s = lax.dot_general(q, k, (((1,), (1,)), ((), ())),
                            preferred_element_type=jnp.float32)
        m = jnp.max(s, axis=-1, keepdims=True)
        e = jnp.exp(s - m)
        denom = jnp.sum(e, axis=-1, keepdims=True)
        lse = m + jnp.log(denom)
        o = lax.dot_general(e.astype(jnp.bfloat16), v.astype(jnp.bfloat16),
                            (((1,), (0,)), ((), ())),
                            preferred_element_type=jnp.float32) / denom
        o_ref[sl, :] = o + jnp.where(lane == DH, lse, 0.0)


def _bucket_attention(sqk, sv):
    nblk = TBL // ATT_ROWS
    return pl.pallas_call(
        _attn_body,
        grid=(nblk,),
        in_specs=[
            pl.BlockSpec((ATT_ROWS, PADW), lambda g: (g, 0)),
            pl.BlockSpec((ATT_ROWS, PADW), lambda g: (g, 0)),
        ],
        out_specs=pl.BlockSpec((ATT_ROWS, EXTW), lambda g: (g, 0)),
        out_shape=jax.ShapeDtypeStruct((TBL, EXTW), jnp.float32),
    )(sqk, sv)


@functools.cache
def _sc_scatter_kernel():
    @functools.partial(
        pl.kernel,
        out_type=jax.ShapeDtypeStruct((TBL, EXTW), jnp.float32),
        mesh=_sc_mesh(),
        scratch_types=[
            pltpu.VMEM((CHUNK,), jnp.int32),
            pltpu.VMEM((CHUNK, EXTW), jnp.float32),
            pltpu.SemaphoreType.DMA,
        ],
        cost_estimate=pl.CostEstimate(
            flops=0, transcendentals=0,
            bytes_accessed=2 * TBL * EXTW * 4 + TBL * 4),
    )
    def body(idx_hbm, src_hbm, out_hbm, idx_v, buf, sem):
        wid = lax.axis_index("s") * SC_CORES + lax.axis_index("c")

        def step(i, carry):
            base = wid * ROWS_PER_W + i * CHUNK
            pltpu.sync_copy(idx_hbm.at[pl.ds(base, CHUNK)], idx_v)
            pltpu.sync_copy(src_hbm.at[pl.ds(base, CHUNK)], buf)
            pltpu.async_copy(buf, out_hbm.at[idx_v], sem).wait()
            return carry

        lax.fori_loop(0, NCHUNKS, step, 0)

    return body


def _combine_body(e0_ref, e1_ref, w_ref, o_ref):
    h = pl.program_id(2)
    e0 = e0_ref[0]
    e1 = e1_ref[0]
    lane = lax.broadcasted_iota(jnp.int32, (BN, EXTW), 1)
    msk = jnp.where(lane == DH, 1.0, 0.0)
    l0 = jnp.sum(e0 * msk, axis=1, keepdims=True)
    l1 = jnp.sum(e1 * msk, axis=1, keepdims=True)
    m = jnp.maximum(l0, l1)
    a0 = jnp.exp(l0 - m)
    a1 = jnp.exp(l1 - m)
    inv = 1.0 / (a0 + a1)
    comb = e0 * (a0 * inv) + e1 * (a1 * inv)
    acc = jnp.dot(comb.astype(jnp.bfloat16), w_ref[0].astype(jnp.bfloat16),
                  preferred_element_type=jnp.float32)

    @pl.when(h == 0)
    def _():
        o_ref[0] = acc

    @pl.when(h > 0)
    def _():
        o_ref[0] += acc


def _combine(e0, e1, Wout_pad):
    espec = pl.BlockSpec((1, BN, EXTW), lambda b, nb, h: (b * HEADS + h, nb, 0))
    return pl.pallas_call(
        _combine_body,
        grid=(B, NB, HEADS),
        in_specs=[
            espec, espec,
            pl.BlockSpec((1, EXTW, CH), lambda b, nb, h: (h, 0, 0)),
        ],
        out_specs=pl.BlockSpec((1, BN, CH), lambda b, nb, h: (b, nb, 0)),
        out_shape=jax.ShapeDtypeStruct((B, N, CH), jnp.float32),
    )(e0.reshape(BS, N, EXTW), e1.reshape(BS, N, EXTW), Wout_pad)


def kernel(input, Wq, Wk, Wv, Wout, alpha, beta):
    inp = input
    e_h = CH // HEADS
    x_hash = inp.reshape(B, N, HEADS, e_h).transpose(0, 2, 1, 3).reshape(BS, N, e_h)
    x_norms = jnp.linalg.norm(x_hash, axis=-1, keepdims=True)
    MX = jnp.max(x_norms, axis=-2, keepdims=True)
    ext = jnp.sqrt(jnp.maximum(MX ** 2 - x_norms ** 2, 0.0))
    Xs = jnp.concatenate([x_hash, ext, jnp.zeros_like(ext)], axis=-1)
    proj = (Xs @ alpha + beta) / 1.0
    x_hashed = jnp.transpose(proj, (2, 0, 1))

    def _w3(W):
        W3 = W.reshape(CH, HEADS, DH).transpose(1, 0, 2)
        return jnp.concatenate(
            [W3, jnp.zeros((HEADS, CH, PADW - DH), W.dtype)], axis=2)
    qkt, vt = _qkv_tables(inp, _w3(Wq), _w3(Wk), _w3(Wv))
    qkt = qkt.reshape(TBL, PADW)
    vt = vt.reshape(TBL, PADW)

    head_off = (jnp.arange(BS, dtype=jnp.int32) * N)[:, None]
    gather = _sc_gather_kernel()
    scatter = _sc_scatter_kernel()

    e_rounds = []
    for r in range(NR):
        pos_r = jnp.argsort(x_hashed[r], axis=-1).astype(jnp.int32)
        idx_r = (pos_r + head_off).reshape(-1)
        sqk, sv = gather(idx_r, qkt, vt)
        bo = _bucket_attention(sqk, sv)
        e_rounds.append(scatter(idx_r, bo))

    Wout_pad = jnp.concatenate(
        [Wout.reshape(HEADS, DH, CH),
         jnp.zeros((HEADS, EXTW - DH, CH), Wout.dtype)], axis=1)
    return _combine(e_rounds[0], e_rounds[1], Wout_pad)


# speedup vs baseline: 1.0364x; 1.0364x over previous
import functools

import jax
import jax.numpy as jnp
from jax import lax
from jax.experimental import pallas as pl
from jax.experimental.pallas import tpu as pltpu
from jax.experimental.pallas import tpu_sc as plsc

B = 2
N = 4096
CH = 256
HEADS = 8
NR = 2
PATCH = 256
DH = 96
PADW = 128
EXTW = 128
BS = B * HEADS
TBL = BS * N

SC_CORES = 2
SC_SUBCORES = 16
NW = SC_CORES * SC_SUBCORES
CHUNK = 128
ROWS_PER_W = TBL // NW
NCHUNKS = ROWS_PER_W // CHUNK

BN = 512
NB = N // BN
ATT_B = 2
ATT_ROWS = ATT_B * PATCH


@functools.cache
def _sc_mesh():
    return plsc.VectorSubcoreMesh(
        core_axis_name="c", subcore_axis_name="s",
        num_cores=SC_CORES, num_subcores=SC_SUBCORES)


def _qkv_body(x_ref, wq_ref, wk_ref, wv_ref, qk_ref, v_ref):
    x = x_ref[0]
    xb = x.astype(jnp.bfloat16)
    q = jnp.dot(xb, wq_ref[0].astype(jnp.bfloat16),
                preferred_element_type=jnp.float32)
    k = jnp.dot(xb, wk_ref[0].astype(jnp.bfloat16),
                preferred_element_type=jnp.float32)
    qu = lax.bitcast_convert_type(q.astype(jnp.bfloat16), jnp.uint16)
    ku = lax.bitcast_convert_type(k.astype(jnp.bfloat16), jnp.uint16)
    word = qu.astype(jnp.uint32) | (ku.astype(jnp.uint32) << 16)
    qk_ref[0] = lax.bitcast_convert_type(word, jnp.int32)
    v_ref[0] = jnp.dot(x, wv_ref[0], preferred_element_type=jnp.float32)


def _qkv_tables(inp, Wq3, Wk3, Wv3):
    grid = (B, NB, HEADS)
    wspec = pl.BlockSpec((1, CH, PADW), lambda b, nb, h: (h, 0, 0))
    return pl.pallas_call(
        _qkv_body,
        grid=grid,
        in_specs=[
            pl.BlockSpec((1, BN, CH), lambda b, nb, h: (b, nb, 0)),
            wspec, wspec, wspec,
        ],
        out_specs=[
            pl.BlockSpec((1, BN, PADW), lambda b, nb, h: (b * HEADS + h, nb, 0)),
            pl.BlockSpec((1, BN, PADW), lambda b, nb, h: (b * HEADS + h, nb, 0)),
        ],
        out_shape=[
            jax.ShapeDtypeStruct((BS, N, PADW), jnp.int32),
            jax.ShapeDtypeStruct((BS, N, PADW), jnp.float32),
        ],
    )(inp, Wq3, Wk3, Wv3)


@functools.cache
def _sc_gather_kernel():
    @functools.partial(
        pl.kernel,
        out_type=[
            jax.ShapeDtypeStruct((TBL, PADW), jnp.int32),
            jax.ShapeDtypeStruct((TBL, PADW), jnp.float32),
        ],
        mesh=_sc_mesh(),
        scratch_types=[
            pltpu.VMEM((CHUNK,), jnp.int32),
            pltpu.VMEM((CHUNK, PADW), jnp.int32),
            pltpu.VMEM((CHUNK, PADW), jnp.float32),
            pltpu.SemaphoreType.DMA,
        ],
        cost_estimate=pl.CostEstimate(
            flops=0, transcendentals=0,
            bytes_accessed=2 * TBL * PADW * 8 + TBL * 4),
    )
    def body(idx_hbm, qkt_hbm, vt_hbm, sqk_hbm, sv_hbm,
             idx_v, bqk, bv, sem):
        wid = lax.axis_index("s") * SC_CORES + lax.axis_index("c")

        def step(i, carry):
            base = wid * ROWS_PER_W + i * CHUNK
            pltpu.sync_copy(idx_hbm.at[pl.ds(base, CHUNK)], idx_v)
            cqk = pltpu.async_copy(qkt_hbm.at[idx_v], bqk, sem)
            cv = pltpu.async_copy(vt_hbm.at[idx_v], bv, sem)
            cqk.wait()
            cv.wait()
            pltpu.sync_copy(bqk, sqk_hbm.at[pl.ds(base, CHUNK)])
            pltpu.sync_copy(bv, sv_hbm.at[pl.ds(base, CHUNK)])
            return carry

        lax.fori_loop(0, NCHUNKS, step, 0)

    return body


def _attn_body(qk_ref, v_ref, o_ref):
    lane = lax.broadcasted_iota(jnp.int32, (PATCH, EXTW), 1)
    for t in range(ATT_B):
        sl = pl.ds(t * PATCH, PATCH)
        w = lax.bitcast_convert_type(qk_ref[sl, :], jnp.uint32)
        q = lax.bitcast_convert_type((w & 0xFFFF).astype(jnp.uint16),
                                     jnp.bfloat16)
        k = lax.bitcast_convert_type((w >> 16).astype(jnp.uint16),
                                     jnp.bfloat16)
        v = v_ref[sl, :]
        s = lax.dot_general(q, k, (((1,), (1,)), ((), ())),
                            preferred_element_type=jnp.float32)
        m = jnp.max(s, axis=-1, keepdims=True)
        e = jnp.exp(s - m)
        denom = jnp.sum(e, axis=-1, keepdims=True)
        lse = m + jnp.log(denom)
        o = lax.dot_general(e, v, (((1,), (0,)), ((), ())),
                            preferred_element_type=jnp.float32) / denom
        o_ref[sl, :] = o + jnp.where(lane == DH, lse, 0.0)


def _bucket_attention(sqk, sv):
    nblk = TBL // ATT_ROWS
    return pl.pallas_call(
        _attn_body,
        grid=(nblk,),
        in_specs=[
            pl.BlockSpec((ATT_ROWS, PADW), lambda g: (g, 0)),
            pl.BlockSpec((ATT_ROWS, PADW), lambda g: (g, 0)),
        ],
        out_specs=pl.BlockSpec((ATT_ROWS, EXTW), lambda g: (g, 0)),
        out_shape=jax.ShapeDtypeStruct((TBL, EXTW), jnp.float32),
    )(sqk, sv)


@functools.cache
def _sc_scatter_kernel():
    @functools.partial(
        pl.kernel,
        out_type=jax.ShapeDtypeStruct((TBL, EXTW), jnp.float32),
        mesh=_sc_mesh(),
        scratch_types=[
            pltpu.VMEM((CHUNK,), jnp.int32),
            pltpu.VMEM((CHUNK, EXTW), jnp.float32),
            pltpu.SemaphoreType.DMA,
        ],
        cost_estimate=pl.CostEstimate(
            flops=0, transcendentals=0,
            bytes_accessed=2 * TBL * EXTW * 4 + TBL * 4),
    )
    def body(idx_hbm, src_hbm, out_hbm, idx_v, buf, sem):
        wid = lax.axis_index("s") * SC_CORES + lax.axis_index("c")

        def step(i, carry):
            base = wid * ROWS_PER_W + i * CHUNK
            pltpu.sync_copy(idx_hbm.at[pl.ds(base, CHUNK)], idx_v)
            pltpu.sync_copy(src_hbm.at[pl.ds(base, CHUNK)], buf)
            pltpu.async_copy(buf, out_hbm.at[idx_v], sem).wait()
            return carry

        lax.fori_loop(0, NCHUNKS, step, 0)

    return body


def _combine_body(e0_ref, e1_ref, w_ref, o_ref):
    h = pl.program_id(2)
    e0 = e0_ref[0]
    e1 = e1_ref[0]
    lane = lax.broadcasted_iota(jnp.int32, (BN, EXTW), 1)
    msk = jnp.where(lane == DH, 1.0, 0.0)
    l0 = jnp.sum(e0 * msk, axis=1, keepdims=True)
    l1 = jnp.sum(e1 * msk, axis=1, keepdims=True)
    m = jnp.maximum(l0, l1)
    a0 = jnp.exp(l0 - m)
    a1 = jnp.exp(l1 - m)
    inv = 1.0 / (a0 + a1)
    comb = e0 * (a0 * inv) + e1 * (a1 * inv)
    acc = jnp.dot(comb, w_ref[0], preferred_element_type=jnp.float32)

    @pl.when(h == 0)
    def _():
        o_ref[0] = acc

    @pl.when(h > 0)
    def _():
        o_ref[0] += acc


def _combine(e0, e1, Wout_pad):
    espec = pl.BlockSpec((1, BN, EXTW), lambda b, nb, h: (b * HEADS + h, nb, 0))
    return pl.pallas_call(
        _combine_body,
        grid=(B, NB, HEADS),
        in_specs=[
            espec, espec,
            pl.BlockSpec((1, EXTW, CH), lambda b, nb, h: (h, 0, 0)),
        ],
        out_specs=pl.BlockSpec((1, BN, CH), lambda b, nb, h: (b, nb, 0)),
        out_shape=jax.ShapeDtypeStruct((B, N, CH), jnp.float32),
    )(e0.reshape(BS, N, EXTW), e1.reshape(BS, N, EXTW), Wout_pad)


def kernel(input, Wq, Wk, Wv, Wout, alpha, beta):
    inp = input
    e_h = CH // HEADS
    x_hash = inp.reshape(B, N, HEADS, e_h).transpose(0, 2, 1, 3).reshape(BS, N, e_h)
    x_norms = jnp.linalg.norm(x_hash, axis=-1, keepdims=True)
    MX = jnp.max(x_norms, axis=-2, keepdims=True)
    ext = jnp.sqrt(jnp.maximum(MX ** 2 - x_norms ** 2, 0.0))
    Xs = jnp.concatenate([x_hash, ext, jnp.zeros_like(ext)], axis=-1)
    proj = (Xs @ alpha + beta) / 1.0
    x_hashed = jnp.transpose(proj, (2, 0, 1))

    def _w3(W):
        W3 = W.reshape(CH, HEADS, DH).transpose(1, 0, 2)
        return jnp.concatenate(
            [W3, jnp.zeros((HEADS, CH, PADW - DH), W.dtype)], axis=2)
    qkt, vt = _qkv_tables(inp, _w3(Wq), _w3(Wk), _w3(Wv))
    qkt = qkt.reshape(TBL, PADW)
    vt = vt.reshape(TBL, PADW)

    head_off = (jnp.arange(BS, dtype=jnp.int32) * N)[:, None]
    gather = _sc_gather_kernel()
    scatter = _sc_scatter_kernel()

    e_rounds = []
    for r in range(NR):
        pos_r = jnp.argsort(x_hashed[r], axis=-1).astype(jnp.int32)
        idx_r = (pos_r + head_off).reshape(-1)
        sqk, sv = gather(idx_r, qkt, vt)
        bo = _bucket_attention(sqk, sv)
        e_rounds.append(scatter(idx_r, bo))

    Wout_pad = jnp.concatenate(
        [Wout.reshape(HEADS, DH, CH),
         jnp.zeros((HEADS, EXTW - DH, CH), Wout.dtype)], axis=1)
    return _combine(e_rounds[0], e_rounds[1], Wout_pad)


# speedup vs baseline: 1.3505x; 1.3030x over previous
import functools

import jax
import jax.numpy as jnp
from jax import lax
from jax.experimental import pallas as pl
from jax.experimental.pallas import tpu as pltpu
from jax.experimental.pallas import tpu_sc as plsc

B = 2
N = 4096
CH = 256
HEADS = 8
NR = 2
PATCH = 256
DH = 96
PADW = 128
EXTW = 128
BS = B * HEADS
TBL = BS * N

SC_CORES = 2
SC_SUBCORES = 16
NW = SC_CORES * SC_SUBCORES
CHUNK = 128
TBLH = TBL // 2
ROWS_PER_W = TBLH // NW
NCHUNKS = ROWS_PER_W // CHUNK

BN = 1024
NB = N // BN
BNC = 512
NBC = N // BNC
ATT_B = 4
ATT_ROWS = ATT_B * PATCH


@functools.cache
def _sc_mesh():
    return plsc.VectorSubcoreMesh(
        core_axis_name="c", subcore_axis_name="s",
        num_cores=SC_CORES, num_subcores=SC_SUBCORES)


def _qkv_body(x_ref, wq_ref, wk_ref, wv_ref, qk_ref, v_ref):
    x = x_ref[0]
    xb = x.astype(jnp.bfloat16)
    q = jnp.dot(xb, wq_ref[0], preferred_element_type=jnp.float32)
    k = jnp.dot(xb, wk_ref[0], preferred_element_type=jnp.float32)
    qu = lax.bitcast_convert_type(q.astype(jnp.bfloat16), jnp.uint16)
    ku = lax.bitcast_convert_type(k.astype(jnp.bfloat16), jnp.uint16)
    word = qu.astype(jnp.uint32) | (ku.astype(jnp.uint32) << 16)
    qk_ref[0] = lax.bitcast_convert_type(word, jnp.int32)
    v_ref[0] = jnp.dot(x, wv_ref[0], preferred_element_type=jnp.float32)


def _qkv_tables(inp, Wq3, Wk3, Wv3):
    grid = (B, NB, HEADS)
    wspec = pl.BlockSpec((1, CH, PADW), lambda b, nb, h: (h, 0, 0))
    return pl.pallas_call(
        _qkv_body,
        grid=grid,
        in_specs=[
            pl.BlockSpec((1, BN, CH), lambda b, nb, h: (b, nb, 0)),
            wspec, wspec, wspec,
        ],
        out_specs=[
            pl.BlockSpec((1, BN, PADW), lambda b, nb, h: (b * HEADS + h, nb, 0)),
            pl.BlockSpec((1, BN, PADW), lambda b, nb, h: (b * HEADS + h, nb, 0)),
        ],
        out_shape=[
            jax.ShapeDtypeStruct((BS, N, PADW), jnp.int32),
            jax.ShapeDtypeStruct((BS, N, PADW), jnp.float32),
        ],
    )(inp, Wq3, Wk3, Wv3)


@functools.cache
def _sc_gather_kernel():
    @functools.partial(
        pl.kernel,
        out_type=[
            jax.ShapeDtypeStruct((TBLH, PADW), jnp.int32),
            jax.ShapeDtypeStruct((TBLH, PADW), jnp.float32),
        ],
        mesh=_sc_mesh(),
        scratch_types=[
            pltpu.VMEM((CHUNK,), jnp.int32),
            pltpu.VMEM((CHUNK, PADW), jnp.int32),
            pltpu.VMEM((CHUNK, PADW), jnp.float32),
            pltpu.SemaphoreType.DMA,
        ],
        cost_estimate=pl.CostEstimate(
            flops=0, transcendentals=0,
            bytes_accessed=2 * TBLH * PADW * 8 + TBLH * 4),
    )
    def body(idx_hbm, qkt_hbm, vt_hbm, sqk_hbm, sv_hbm,
             idx_v, bqk, bv, sem):
        wid = lax.axis_index("s") * SC_CORES + lax.axis_index("c")

        def step(i, carry):
            base = wid * ROWS_PER_W + i * CHUNK
            pltpu.sync_copy(idx_hbm.at[pl.ds(base, CHUNK)], idx_v)
            cqk = pltpu.async_copy(qkt_hbm.at[idx_v], bqk, sem)
            cv = pltpu.async_copy(vt_hbm.at[idx_v], bv, sem)
            cqk.wait()
            cv.wait()
            pltpu.sync_copy(bqk, sqk_hbm.at[pl.ds(base, CHUNK)])
            pltpu.sync_copy(bv, sv_hbm.at[pl.ds(base, CHUNK)])
            return carry

        lax.fori_loop(0, NCHUNKS, step, 0)

    return body


def _attn_body(qk_ref, v_ref, o_ref):
    lane = lax.broadcasted_iota(jnp.int32, (PATCH, EXTW), 1)
    for t in range(ATT_B):
        sl = pl.ds(t * PATCH, PATCH)
        w = lax.bitcast_convert_type(qk_ref[sl, :], jnp.uint32)
        q = lax.bitcast_convert_type((w & 0xFFFF).astype(jnp.uint16),
                                     jnp.bfloat16)
        k = lax.bitcast_convert_type((w >> 16).astype(jnp.uint16),
                                     jnp.bfloat16)
        v = v_ref[sl, :]
        s = lax.dot_general(q, k, (((1,), (1,)), ((), ())),
                            preferred_element_type=jnp.float32)
        m = jnp.max(s, axis=-1, keepdims=True)
        e = jnp.exp(s - m)
        denom = jnp.sum(e, axis=-1, keepdims=True)
        lse = m + jnp.log(denom)
        o = lax.dot_general(e, v, (((1,), (0,)), ((), ())),
                            preferred_element_type=jnp.float32) / denom
        o_ref[sl, :] = o + jnp.where(lane == DH, lse, 0.0)


def _bucket_attention(sqk, sv):
    nblk = TBLH // ATT_ROWS
    return pl.pallas_call(
        _attn_body,
        grid=(nblk,),
        in_specs=[
            pl.BlockSpec((ATT_ROWS, PADW), lambda g: (g, 0)),
            pl.BlockSpec((ATT_ROWS, PADW), lambda g: (g, 0)),
        ],
        out_specs=pl.BlockSpec((ATT_ROWS, EXTW), lambda g: (g, 0)),
        out_shape=jax.ShapeDtypeStruct((TBLH, EXTW), jnp.float32),
    )(sqk, sv)


@functools.cache
def _sc_scatter_kernel():
    @functools.partial(
        pl.kernel,
        out_type=jax.ShapeDtypeStruct((TBLH, EXTW), jnp.float32),
        mesh=_sc_mesh(),
        scratch_types=[
            pltpu.VMEM((CHUNK,), jnp.int32),
            pltpu.VMEM((CHUNK, EXTW), jnp.float32),
            pltpu.SemaphoreType.DMA,
        ],
        cost_estimate=pl.CostEstimate(
            flops=0, transcendentals=0,
            bytes_accessed=2 * TBLH * EXTW * 4 + TBLH * 4),
    )
    def body(idx_hbm, src_hbm, out_hbm, idx_v, buf, sem):
        wid = lax.axis_index("s") * SC_CORES + lax.axis_index("c")

        def step(i, carry):
            base = wid * ROWS_PER_W + i * CHUNK
            pltpu.sync_copy(idx_hbm.at[pl.ds(base, CHUNK)], idx_v)
            pltpu.sync_copy(src_hbm.at[pl.ds(base, CHUNK)], buf)
            pltpu.async_copy(buf, out_hbm.at[idx_v], sem).wait()
            return carry

        lax.fori_loop(0, NCHUNKS, step, 0)

    return body


def _combine_body(e0_ref, e1_ref, w_ref, o_ref):
    h = pl.program_id(1)
    e0 = e0_ref[0]
    e1 = e1_ref[0]
    lane = lax.broadcasted_iota(jnp.int32, (BNC, EXTW), 1)
    msk = jnp.where(lane == DH, 1.0, 0.0)
    l0 = jnp.sum(e0 * msk, axis=1, keepdims=True)
    l1 = jnp.sum(e1 * msk, axis=1, keepdims=True)
    m = jnp.maximum(l0, l1)
    a0 = jnp.exp(l0 - m)
    a1 = jnp.exp(l1 - m)
    inv = 1.0 / (a0 + a1)
    comb = e0 * (a0 * inv) + e1 * (a1 * inv)
    acc = jnp.dot(comb, w_ref[0], preferred_element_type=jnp.float32)

    @pl.when(h == 0)
    def _():
        o_ref[...] = acc

    @pl.when(h > 0)
    def _():
        o_ref[...] += acc


def _combine_half(e0, e1, Wout_pad):
    espec = pl.BlockSpec((1, BNC, EXTW), lambda nb, h: (h, nb, 0))
    return pl.pallas_call(
        _combine_body,
        grid=(NBC, HEADS),
        in_specs=[
            espec, espec,
            pl.BlockSpec((1, EXTW, CH), lambda nb, h: (h, 0, 0)),
        ],
        out_specs=pl.BlockSpec((BNC, CH), lambda nb, h: (nb, 0)),
        out_shape=jax.ShapeDtypeStruct((N, CH), jnp.float32),
    )(e0.reshape(HEADS, N, EXTW), e1.reshape(HEADS, N, EXTW), Wout_pad)


def kernel(input, Wq, Wk, Wv, Wout, alpha, beta):
    inp = input
    e_h = CH // HEADS
    x_hash = inp.reshape(B, N, HEADS, e_h).transpose(0, 2, 1, 3).reshape(BS, N, e_h)
    x_norms = jnp.linalg.norm(x_hash, axis=-1, keepdims=True)
    MX = jnp.max(x_norms, axis=-2, keepdims=True)
    ext = jnp.sqrt(jnp.maximum(MX ** 2 - x_norms ** 2, 0.0))
    Xs = jnp.concatenate([x_hash, ext, jnp.zeros_like(ext)], axis=-1)
    proj = (Xs @ alpha + beta) / 1.0
    x_hashed = jnp.transpose(proj, (2, 0, 1))

    def _w3(W):
        W3 = W.reshape(CH, HEADS, DH).transpose(1, 0, 2)
        return jnp.concatenate(
            [W3, jnp.zeros((HEADS, CH, PADW - DH), W.dtype)], axis=2)
    qkt, vt = _qkv_tables(inp, _w3(Wq).astype(jnp.bfloat16),
                          _w3(Wk).astype(jnp.bfloat16), _w3(Wv))
    qkt = qkt.reshape(TBL, PADW)
    vt = vt.reshape(TBL, PADW)

    gidx_off = (jnp.arange(BS, dtype=jnp.int32) * N).reshape(B, HEADS)[:, :, None]
    sidx_off = (jnp.arange(HEADS, dtype=jnp.int32) * N)[None, :, None]
    gather = _sc_gather_kernel()
    scatter = _sc_scatter_kernel()

    pos = [jnp.argsort(x_hashed[r], axis=-1).astype(jnp.int32).reshape(B, HEADS, N)
           for r in range(NR)]
    e_halves = [[None] * B for _ in range(NR)]
    for r in range(NR):
        for b in range(B):
            gidx = (pos[r][b] + gidx_off[b]).reshape(-1)
            sidx = (pos[r][b] + sidx_off[0]).reshape(-1)
            sqk, sv = gather(gidx, qkt, vt)
            bo = _bucket_attention(sqk, sv)
            e_halves[r][b] = scatter(sidx, bo)

    Wout_pad = jnp.concatenate(
        [Wout.reshape(HEADS, DH, CH),
         jnp.zeros((HEADS, EXTW - DH, CH), Wout.dtype)], axis=1)
    outs = [_combine_half(e_halves[0][b], e_halves[1][b], Wout_pad)
            for b in range(B)]
    return jnp.stack(outs, axis=0)


# speedup vs baseline: 1.4239x; 1.0544x over previous
import functools

import jax
import jax.numpy as jnp
from jax import lax
from jax.experimental import pallas as pl
from jax.experimental.pallas import tpu as pltpu
from jax.experimental.pallas import tpu_sc as plsc

B = 2
N = 4096
CH = 256
HEADS = 8
NR = 2
PATCH = 256
DH = 96
PADW = 128
EXTW = 128
BS = B * HEADS
TBL = BS * N

SC_CORES = 2
SC_SUBCORES = 16
NW = SC_CORES * SC_SUBCORES
CHUNK = 128
TBLH = TBL // 2
ROWS_PER_W = TBLH // NW
NCHUNKS = ROWS_PER_W // CHUNK

BN = 1024
NB = N // BN
BNC = 512
NBC = N // BNC
ATT_B = 4
ATT_ROWS = ATT_B * PATCH


@functools.cache
def _sc_mesh():
    return plsc.VectorSubcoreMesh(
        core_axis_name="c", subcore_axis_name="s",
        num_cores=SC_CORES, num_subcores=SC_SUBCORES)


def _qkv_body(x_ref, wq_ref, wk_ref, wv_ref, qk_ref, v_ref):
    x = x_ref[...]
    xb = x.astype(jnp.bfloat16)
    q = jnp.dot(xb, wq_ref[0], preferred_element_type=jnp.float32)
    k = jnp.dot(xb, wk_ref[0], preferred_element_type=jnp.float32)
    qu = lax.bitcast_convert_type(q.astype(jnp.bfloat16), jnp.uint16)
    ku = lax.bitcast_convert_type(k.astype(jnp.bfloat16), jnp.uint16)
    word = qu.astype(jnp.uint32) | (ku.astype(jnp.uint32) << 16)
    qk_ref[0] = lax.bitcast_convert_type(word, jnp.int32)
    v_ref[0] = jnp.dot(x, wv_ref[0], preferred_element_type=jnp.float32)


def _qkv_tables_b(inp_b, Wq3, Wk3, Wv3):
    wspec = pl.BlockSpec((1, CH, PADW), lambda nb, h: (h, 0, 0))
    return pl.pallas_call(
        _qkv_body,
        grid=(NB, HEADS),
        in_specs=[
            pl.BlockSpec((BN, CH), lambda nb, h: (nb, 0)),
            wspec, wspec, wspec,
        ],
        out_specs=[
            pl.BlockSpec((1, BN, PADW), lambda nb, h: (h, nb, 0)),
            pl.BlockSpec((1, BN, PADW), lambda nb, h: (h, nb, 0)),
        ],
        out_shape=[
            jax.ShapeDtypeStruct((HEADS, N, PADW), jnp.int32),
            jax.ShapeDtypeStruct((HEADS, N, PADW), jnp.float32),
        ],
    )(inp_b, Wq3, Wk3, Wv3)


@functools.cache
def _sc_gather_kernel():
    @functools.partial(
        pl.kernel,
        out_type=[
            jax.ShapeDtypeStruct((TBLH, PADW), jnp.int32),
            jax.ShapeDtypeStruct((TBLH, PADW), jnp.float32),
        ],
        mesh=_sc_mesh(),
        scratch_types=[
            pltpu.VMEM((CHUNK,), jnp.int32),
            pltpu.VMEM((CHUNK, PADW), jnp.int32),
            pltpu.VMEM((CHUNK, PADW), jnp.float32),
            pltpu.SemaphoreType.DMA,
        ],
        cost_estimate=pl.CostEstimate(
            flops=0, transcendentals=0,
            bytes_accessed=2 * TBLH * PADW * 8 + TBLH * 4),
    )
    def body(idx_hbm, qkt_hbm, vt_hbm, sqk_hbm, sv_hbm,
             idx_v, bqk, bv, sem):
        wid = lax.axis_index("s") * SC_CORES + lax.axis_index("c")

        def step(i, carry):
            base = wid * ROWS_PER_W + i * CHUNK
            pltpu.sync_copy(idx_hbm.at[pl.ds(base, CHUNK)], idx_v)
            cqk = pltpu.async_copy(qkt_hbm.at[idx_v], bqk, sem)
            cv = pltpu.async_copy(vt_hbm.at[idx_v], bv, sem)
            cqk.wait()
            cv.wait()
            pltpu.sync_copy(bqk, sqk_hbm.at[pl.ds(base, CHUNK)])
            pltpu.sync_copy(bv, sv_hbm.at[pl.ds(base, CHUNK)])
            return carry

        lax.fori_loop(0, NCHUNKS, step, 0)

    return body


def _attn_body(qk_ref, v_ref, o_ref):
    lane = lax.broadcasted_iota(jnp.int32, (PATCH, EXTW), 1)
    for t in range(ATT_B):
        sl = pl.ds(t * PATCH, PATCH)
        w = lax.bitcast_convert_type(qk_ref[sl, :], jnp.uint32)
        q = lax.bitcast_convert_type((w & 0xFFFF).astype(jnp.uint16),
                                     jnp.bfloat16)
        k = lax.bitcast_convert_type((w >> 16).astype(jnp.uint16),
                                     jnp.bfloat16)
        v = v_ref[sl, :]
        s = lax.dot_general(q, k, (((1,), (1,)), ((), ())),
                            preferred_element_type=jnp.float32)
        m = jnp.max(s, axis=-1, keepdims=True)
        e = jnp.exp(s - m)
        denom = jnp.sum(e, axis=-1, keepdims=True)
        lse = m + jnp.log(denom)
        o = lax.dot_general(e, v, (((1,), (0,)), ((), ())),
                            preferred_element_type=jnp.float32) / denom
        o_ref[sl, :] = o + jnp.where(lane == DH, lse, 0.0)


def _bucket_attention(sqk, sv):
    nblk = TBLH // ATT_ROWS
    return pl.pallas_call(
        _attn_body,
        grid=(nblk,),
        in_specs=[
            pl.BlockSpec((ATT_ROWS, PADW), lambda g: (g, 0)),
            pl.BlockSpec((ATT_ROWS, PADW), lambda g: (g, 0)),
        ],
        out_specs=pl.BlockSpec((ATT_ROWS, EXTW), lambda g: (g, 0)),
        out_shape=jax.ShapeDtypeStruct((TBLH, EXTW), jnp.float32),
    )(sqk, sv)


@functools.cache
def _sc_scatter_kernel():
    @functools.partial(
        pl.kernel,
        out_type=jax.ShapeDtypeStruct((TBLH, EXTW), jnp.float32),
        mesh=_sc_mesh(),
        scratch_types=[
            pltpu.VMEM((CHUNK,), jnp.int32),
            pltpu.VMEM((CHUNK, EXTW), jnp.float32),
            pltpu.SemaphoreType.DMA,
        ],
        cost_estimate=pl.CostEstimate(
            flops=0, transcendentals=0,
            bytes_accessed=2 * TBLH * EXTW * 4 + TBLH * 4),
    )
    def body(idx_hbm, src_hbm, out_hbm, idx_v, buf, sem):
        wid = lax.axis_index("s") * SC_CORES + lax.axis_index("c")

        def step(i, carry):
            base = wid * ROWS_PER_W + i * CHUNK
            pltpu.sync_copy(idx_hbm.at[pl.ds(base, CHUNK)], idx_v)
            pltpu.sync_copy(src_hbm.at[pl.ds(base, CHUNK)], buf)
            pltpu.async_copy(buf, out_hbm.at[idx_v], sem).wait()
            return carry

        lax.fori_loop(0, NCHUNKS, step, 0)

    return body


def _combine_body(e0_ref, e1_ref, w_ref, o_ref):
    h = pl.program_id(1)
    e0 = e0_ref[0]
    e1 = e1_ref[0]
    lane = lax.broadcasted_iota(jnp.int32, (BNC, EXTW), 1)
    msk = jnp.where(lane == DH, 1.0, 0.0)
    l0 = jnp.sum(e0 * msk, axis=1, keepdims=True)
    l1 = jnp.sum(e1 * msk, axis=1, keepdims=True)
    m = jnp.maximum(l0, l1)
    a0 = jnp.exp(l0 - m)
    a1 = jnp.exp(l1 - m)
    inv = 1.0 / (a0 + a1)
    comb = e0 * (a0 * inv) + e1 * (a1 * inv)
    acc = jnp.dot(comb, w_ref[0], preferred_element_type=jnp.float32)

    @pl.when(h == 0)
    def _():
        o_ref[...] = acc

    @pl.when(h > 0)
    def _():
        o_ref[...] += acc


def _combine_half(e0, e1, Wout_pad):
    espec = pl.BlockSpec((1, BNC, EXTW), lambda nb, h: (h, nb, 0))
    return pl.pallas_call(
        _combine_body,
        grid=(NBC, HEADS),
        in_specs=[
            espec, espec,
            pl.BlockSpec((1, EXTW, CH), lambda nb, h: (h, 0, 0)),
        ],
        out_specs=pl.BlockSpec((BNC, CH), lambda nb, h: (nb, 0)),
        out_shape=jax.ShapeDtypeStruct((N, CH), jnp.float32),
    )(e0.reshape(HEADS, N, EXTW), e1.reshape(HEADS, N, EXTW), Wout_pad)


def kernel(input, Wq, Wk, Wv, Wout, alpha, beta):
    inp = input
    e_h = CH // HEADS
    x_hash = inp.reshape(B, N, HEADS, e_h).transpose(0, 2, 1, 3).reshape(BS, N, e_h)
    x_norms = jnp.linalg.norm(x_hash, axis=-1, keepdims=True)
    MX = jnp.max(x_norms, axis=-2, keepdims=True)
    ext = jnp.sqrt(jnp.maximum(MX ** 2 - x_norms ** 2, 0.0))
    Xs = jnp.concatenate([x_hash, ext, jnp.zeros_like(ext)], axis=-1)
    proj = (Xs @ alpha + beta) / 1.0
    x_hashed = jnp.transpose(proj, (2, 0, 1))

    pos_all = jnp.argsort(x_hashed.reshape(NR * BS, N), axis=-1).astype(jnp.int32)
    pos = pos_all.reshape(NR, B, HEADS, N)

    def _w3(W):
        W3 = W.reshape(CH, HEADS, DH).transpose(1, 0, 2)
        return jnp.concatenate(
            [W3, jnp.zeros((HEADS, CH, PADW - DH), W.dtype)], axis=2)
    wq3 = _w3(Wq).astype(jnp.bfloat16)
    wk3 = _w3(Wk).astype(jnp.bfloat16)
    wv3 = _w3(Wv)

    idx_off = (jnp.arange(HEADS, dtype=jnp.int32) * N)[:, None]
    gather = _sc_gather_kernel()
    scatter = _sc_scatter_kernel()

    tables = []
    e_halves = [[None] * B for _ in range(NR)]
    for b in range(B):
        qkt_b, vt_b = _qkv_tables_b(inp[b], wq3, wk3, wv3)
        tables.append((qkt_b.reshape(TBLH, PADW), vt_b.reshape(TBLH, PADW)))
        for r in range(NR):
            idx = (pos[r][b] + idx_off).reshape(-1)
            sqk, sv = gather(idx, tables[b][0], tables[b][1])
            bo = _bucket_attention(sqk, sv)
            e_halves[r][b] = scatter(idx, bo)

    Wout_pad = jnp.concatenate(
        [Wout.reshape(HEADS, DH, CH),
         jnp.zeros((HEADS, EXTW - DH, CH), Wout.dtype)], axis=1)
    outs = [_combine_half(e_halves[0][b], e_halves[1][b], Wout_pad)
            for b in range(B)]
    return jnp.stack(outs, axis=0)


# speedup vs baseline: 1.4630x; 1.0275x over previous
import functools

import jax
import jax.numpy as jnp
from jax import lax
from jax.experimental import pallas as pl
from jax.experimental.pallas import tpu as pltpu
from jax.experimental.pallas import tpu_sc as plsc

B = 2
N = 4096
CH = 256
HEADS = 8
NR = 2
PATCH = 256
DH = 96
PADW = 128
EXTW = 128
BS = B * HEADS
TBL = BS * N

SC_CORES = 2
SC_SUBCORES = 16
NW = SC_CORES * SC_SUBCORES
CHUNK = 128
TBLH = TBL // 2
ROWS_PER_W = TBLH // NW
NCHUNKS = ROWS_PER_W // CHUNK

BN = 1024
NB = N // BN
BNC = 1024
NBC = N // BNC
ATT_B = 8
ATT_ROWS = ATT_B * PATCH


@functools.cache
def _sc_mesh():
    return plsc.VectorSubcoreMesh(
        core_axis_name="c", subcore_axis_name="s",
        num_cores=SC_CORES, num_subcores=SC_SUBCORES)


def _qkv_body(x_ref, wq_ref, wk_ref, wv_ref, qk_ref, v_ref):
    x = x_ref[...]
    xb = x.astype(jnp.bfloat16)
    q = jnp.dot(xb, wq_ref[0], preferred_element_type=jnp.float32)
    k = jnp.dot(xb, wk_ref[0], preferred_element_type=jnp.float32)
    qu = lax.bitcast_convert_type(q.astype(jnp.bfloat16), jnp.uint16)
    ku = lax.bitcast_convert_type(k.astype(jnp.bfloat16), jnp.uint16)
    word = qu.astype(jnp.uint32) | (ku.astype(jnp.uint32) << 16)
    qk_ref[0] = lax.bitcast_convert_type(word, jnp.int32)
    v_ref[0] = jnp.dot(x, wv_ref[0], preferred_element_type=jnp.float32)


def _qkv_tables_b(inp_b, Wq3, Wk3, Wv3):
    wspec = pl.BlockSpec((1, CH, PADW), lambda nb, h: (h, 0, 0))
    return pl.pallas_call(
        _qkv_body,
        grid=(NB, HEADS),
        in_specs=[
            pl.BlockSpec((BN, CH), lambda nb, h: (nb, 0)),
            wspec, wspec, wspec,
        ],
        out_specs=[
            pl.BlockSpec((1, BN, PADW), lambda nb, h: (h, nb, 0)),
            pl.BlockSpec((1, BN, PADW), lambda nb, h: (h, nb, 0)),
        ],
        out_shape=[
            jax.ShapeDtypeStruct((HEADS, N, PADW), jnp.int32),
            jax.ShapeDtypeStruct((HEADS, N, PADW), jnp.float32),
        ],
    )(inp_b, Wq3, Wk3, Wv3)


@functools.cache
def _sc_gather_kernel():
    @functools.partial(
        pl.kernel,
        out_type=[
            jax.ShapeDtypeStruct((TBLH, PADW), jnp.int32),
            jax.ShapeDtypeStruct((TBLH, PADW), jnp.float32),
        ],
        mesh=_sc_mesh(),
        scratch_types=[
            pltpu.VMEM((CHUNK,), jnp.int32),
            pltpu.VMEM((CHUNK, PADW), jnp.int32),
            pltpu.VMEM((CHUNK, PADW), jnp.float32),
            pltpu.SemaphoreType.DMA,
        ],
        cost_estimate=pl.CostEstimate(
            flops=0, transcendentals=0,
            bytes_accessed=2 * TBLH * PADW * 8 + TBLH * 4),
    )
    def body(idx_hbm, qkt_hbm, vt_hbm, sqk_hbm, sv_hbm,
             idx_v, bqk, bv, sem):
        wid = lax.axis_index("s") * SC_CORES + lax.axis_index("c")

        def step(i, carry):
            base = wid * ROWS_PER_W + i * CHUNK
            pltpu.sync_copy(idx_hbm.at[pl.ds(base, CHUNK)], idx_v)
            cqk = pltpu.async_copy(qkt_hbm.at[idx_v], bqk, sem)
            cv = pltpu.async_copy(vt_hbm.at[idx_v], bv, sem)
            cqk.wait()
            cv.wait()
            pltpu.sync_copy(bqk, sqk_hbm.at[pl.ds(base, CHUNK)])
            pltpu.sync_copy(bv, sv_hbm.at[pl.ds(base, CHUNK)])
            return carry

        lax.fori_loop(0, NCHUNKS, step, 0)

    return body


def _attn_body(qk_ref, v_ref, o_ref):
    lane = lax.broadcasted_iota(jnp.int32, (PATCH, EXTW), 1)
    for t in range(ATT_B):
        sl = pl.ds(t * PATCH, PATCH)
        w = lax.bitcast_convert_type(qk_ref[sl, :], jnp.uint32)
        q = lax.bitcast_convert_type((w & 0xFFFF).astype(jnp.uint16),
                                     jnp.bfloat16)
        k = lax.bitcast_convert_type((w >> 16).astype(jnp.uint16),
                                     jnp.bfloat16)
        v = v_ref[sl, :]
        s = lax.dot_general(q, k, (((1,), (1,)), ((), ())),
                            preferred_element_type=jnp.float32)
        m = jnp.max(s, axis=-1, keepdims=True)
        e = jnp.exp(s - m)
        denom = jnp.sum(e, axis=-1, keepdims=True)
        lse = m + jnp.log(denom)
        o = lax.dot_general(e, v, (((1,), (0,)), ((), ())),
                            preferred_element_type=jnp.float32) / denom
        o_ref[sl, :] = o + jnp.where(lane == DH, lse, 0.0)


def _bucket_attention(sqk, sv):
    nblk = TBLH // ATT_ROWS
    return pl.pallas_call(
        _attn_body,
        grid=(nblk,),
        in_specs=[
            pl.BlockSpec((ATT_ROWS, PADW), lambda g: (g, 0)),
            pl.BlockSpec((ATT_ROWS, PADW), lambda g: (g, 0)),
        ],
        out_specs=pl.BlockSpec((ATT_ROWS, EXTW), lambda g: (g, 0)),
        out_shape=jax.ShapeDtypeStruct((TBLH, EXTW), jnp.float32),
    )(sqk, sv)


@functools.cache
def _sc_scatter_kernel():
    @functools.partial(
        pl.kernel,
        out_type=jax.ShapeDtypeStruct((TBLH, EXTW), jnp.float32),
        mesh=_sc_mesh(),
        scratch_types=[
            pltpu.VMEM((CHUNK,), jnp.int32),
            pltpu.VMEM((CHUNK, EXTW), jnp.float32),
            pltpu.SemaphoreType.DMA,
        ],
        cost_estimate=pl.CostEstimate(
            flops=0, transcendentals=0,
            bytes_accessed=2 * TBLH * EXTW * 4 + TBLH * 4),
    )
    def body(idx_hbm, src_hbm, out_hbm, idx_v, buf, sem):
        wid = lax.axis_index("s") * SC_CORES + lax.axis_index("c")

        def step(i, carry):
            base = wid * ROWS_PER_W + i * CHUNK
            pltpu.sync_copy(idx_hbm.at[pl.ds(base, CHUNK)], idx_v)
            pltpu.sync_copy(src_hbm.at[pl.ds(base, CHUNK)], buf)
            pltpu.async_copy(buf, out_hbm.at[idx_v], sem).wait()
            return carry

        lax.fori_loop(0, NCHUNKS, step, 0)

    return body


def _combine_body(e0_ref, e1_ref, w_ref, o_ref):
    h = pl.program_id(1)
    e0 = e0_ref[0]
    e1 = e1_ref[0]
    lane = lax.broadcasted_iota(jnp.int32, (BNC, EXTW), 1)
    msk = jnp.where(lane == DH, 1.0, 0.0)
    l0 = jnp.sum(e0 * msk, axis=1, keepdims=True)
    l1 = jnp.sum(e1 * msk, axis=1, keepdims=True)
    m = jnp.maximum(l0, l1)
    a0 = jnp.exp(l0 - m)
    a1 = jnp.exp(l1 - m)
    inv = 1.0 / (a0 + a1)
    comb = e0 * (a0 * inv) + e1 * (a1 * inv)
    acc = jnp.dot(comb, w_ref[0], preferred_element_type=jnp.float32)

    @pl.when(h == 0)
    def _():
        o_ref[...] = acc

    @pl.when(h > 0)
    def _():
        o_ref[...] += acc


def _combine_half(e0, e1, Wout_pad):
    espec = pl.BlockSpec((1, BNC, EXTW), lambda nb, h: (h, nb, 0))
    return pl.pallas_call(
        _combine_body,
        grid=(NBC, HEADS),
        in_specs=[
            espec, espec,
            pl.BlockSpec((1, EXTW, CH), lambda nb, h: (h, 0, 0)),
        ],
        out_specs=pl.BlockSpec((BNC, CH), lambda nb, h: (nb, 0)),
        out_shape=jax.ShapeDtypeStruct((N, CH), jnp.float32),
    )(e0.reshape(HEADS, N, EXTW), e1.reshape(HEADS, N, EXTW), Wout_pad)


def kernel(input, Wq, Wk, Wv, Wout, alpha, beta):
    inp = input
    e_h = CH // HEADS
    x_hash = inp.reshape(B, N, HEADS, e_h).transpose(0, 2, 1, 3).reshape(BS, N, e_h)
    x_norms = jnp.linalg.norm(x_hash, axis=-1, keepdims=True)
    MX = jnp.max(x_norms, axis=-2, keepdims=True)
    ext = jnp.sqrt(jnp.maximum(MX ** 2 - x_norms ** 2, 0.0))
    Xs = jnp.concatenate([x_hash, ext, jnp.zeros_like(ext)], axis=-1)
    proj = (Xs @ alpha + beta) / 1.0
    x_hashed = jnp.transpose(proj, (2, 0, 1))

    pos_all = jnp.argsort(x_hashed.reshape(NR * BS, N), axis=-1).astype(jnp.int32)
    pos = pos_all.reshape(NR, B, HEADS, N)

    def _w3(W):
        W3 = W.reshape(CH, HEADS, DH).transpose(1, 0, 2)
        return jnp.concatenate(
            [W3, jnp.zeros((HEADS, CH, PADW - DH), W.dtype)], axis=2)
    wq3 = _w3(Wq).astype(jnp.bfloat16)
    wk3 = _w3(Wk).astype(jnp.bfloat16)
    wv3 = _w3(Wv)

    idx_off = (jnp.arange(HEADS, dtype=jnp.int32) * N)[:, None]
    gather = _sc_gather_kernel()
    scatter = _sc_scatter_kernel()

    tables = []
    e_halves = [[None] * B for _ in range(NR)]
    for b in range(B):
        qkt_b, vt_b = _qkv_tables_b(inp[b], wq3, wk3, wv3)
        tables.append((qkt_b.reshape(TBLH, PADW), vt_b.reshape(TBLH, PADW)))
        for r in range(NR):
            idx = (pos[r][b] + idx_off).reshape(-1)
            sqk, sv = gather(idx, tables[b][0], tables[b][1])
            bo = _bucket_attention(sqk, sv)
            e_halves[r][b] = scatter(idx, bo)

    Wout_pad = jnp.concatenate(
        [Wout.reshape(HEADS, DH, CH),
         jnp.zeros((HEADS, EXTW - DH, CH), Wout.dtype)], axis=1)
    outs = [_combine_half(e_halves[0][b], e_halves[1][b], Wout_pad)
            for b in range(B)]
    return jnp.stack(outs, axis=0)
